# baseline (speedup 1.0000x reference)
import functools

import jax
import jax.numpy as jnp
from jax.experimental import pallas as pl
from jax.experimental.pallas import tpu as pltpu

DEPTH = 8


def _x_copies(x_hbm, x_buf, sem_x, t, nb, jblocks, c):
    i = t // jblocks
    j = t % jblocks
    slot = jax.lax.rem(t, DEPTH)
    return [
        pltpu.make_async_copy(
            x_hbm.at[i, :, pl.ds(j * nb, nb)],
            x_buf.at[slot],
            sem_x.at[slot])
    ]


def _score_copies(score_buf, score_hbm, sem_s, t, nb):
    slot = jax.lax.rem(t, DEPTH)
    qr = nb // 2
    return [
        pltpu.make_async_copy(
            score_buf.at[slot, pl.ds(q * qr, qr), :],
            score_hbm.at[pl.ds(t * nb + q * qr, qr), :],
            sem_s.at[slot])
        for q in range(2)
    ]


def _out_copies(out_buf, out_hbm, sem_o, t, nb, jblocks, c, ydim):
    i = t // jblocks
    j = t % jblocks
    slot = jax.lax.rem(t, DEPTH)
    rows = [(0, c), (c, c + ydim)]
    return [
        pltpu.make_async_copy(
            out_buf.at[slot, pl.ds(r0, rn), :],
            out_hbm.at[i, pl.ds(r0, rn), pl.ds(j * nb, nb)],
            sem_o.at[slot])
        for (r0, rn) in rows
    ]


def _body(x_hbm, ft_hbm, fl_hbm, out_hbm, score_hbm,
          x_buf, score_buf, out_buf, ftv, mn, flv,
          sem_ft, sem_fl, sem_x, sem_s, sem_o,
          *, c, k, ydim, nb, jblocks, nsteps):
    t = pl.program_id(0)
    slot = jax.lax.rem(t, DEPTH)

    @pl.when(t == 0)
    def _init():
        cp_ft = pltpu.make_async_copy(ft_hbm, ftv, sem_ft)
        cp_fl = pltpu.make_async_copy(fl_hbm, flv, sem_fl)
        cp_ft.start()
        cp_fl.start()
        for tt in range(3):
            for cp in _x_copies(x_hbm, x_buf, sem_x, tt, nb, jblocks, c):
                cp.start()
        cp_ft.wait()
        cp_fl.wait()
        ft = ftv[...]
        csq = jnp.sum(ft * ft, axis=0, keepdims=True)
        cinv = 1.0 / jnp.maximum(jnp.sqrt(csq), 1e-12)
        mn[...] = (ft * cinv).astype(jnp.bfloat16)

    @pl.when(jnp.logical_and(t >= 1, t + 2 < nsteps))
    def _prefetch():
        for cp in _x_copies(x_hbm, x_buf, sem_x, t + 2, nb, jblocks, c):
            cp.start()

    @pl.when(t >= DEPTH)
    def _drain():
        for cp in _score_copies(score_buf, score_hbm, sem_s, t - DEPTH, nb):
            cp.wait()
        for cp in _out_copies(out_buf, out_hbm, sem_o, t - DEPTH, nb,
                              jblocks, c, ydim):
            cp.wait()

    @pl.when(t < nsteps)
    def _compute():
        for cp in _x_copies(x_hbm, x_buf, sem_x, t, nb, jblocks, c):
            cp.wait()
        xf = x_buf[slot].astype(jnp.float32)
        ssq = jnp.sum(xf * xf, axis=0, keepdims=True)
        rinv = 1.0 / jnp.maximum(jnp.sqrt(ssq), 1e-12)
        xn_t = (xf * rinv).astype(jnp.bfloat16)
        s = jax.lax.dot_general(
            xn_t, mn[...],
            dimension_numbers=(((0,), (0,)), ((), ())),
            preferred_element_type=jnp.float32)
        score_buf[slot] = s
        p = jnp.exp(s)
        dinv = 1.0 / jnp.sum(p, axis=1, keepdims=True)
        oxy = jax.lax.dot_general(
            p.astype(jnp.bfloat16), flv[...],
            dimension_numbers=(((1,), (0,)), ((), ())),
            preferred_element_type=jnp.float32)
        oxy = oxy * dinv
        out_buf[slot, :c, :] = xf
        out_buf[slot, c:, :] = oxy.T
        for cp in _score_copies(score_buf, score_hbm, sem_s, t, nb):
            cp.start()
        for cp in _out_copies(out_buf, out_hbm, sem_o, t, nb, jblocks,
                              c, ydim):
            cp.start()


def kernel(x, feat_units, label_units):
    b, c, h, w = x.shape
    k, ydim = label_units.shape[0], label_units.shape[1]
    n_per_b = h * w
    nb = 512 if n_per_b % 512 == 0 else n_per_b
    jblocks = n_per_b // nb
    nsteps = b * jblocks

    x3 = x.reshape(b, c, n_per_b).astype(jnp.bfloat16)
    ft = feat_units.T
    fl = jnp.concatenate([feat_units, label_units],
                         axis=1).astype(jnp.bfloat16)

    out3, score = pl.pallas_call(
        functools.partial(_body, c=c, k=k, ydim=ydim, nb=nb,
                          jblocks=jblocks, nsteps=nsteps),
        grid=(nsteps + DEPTH,),
        in_specs=[
            pl.BlockSpec(memory_space=pl.ANY),
            pl.BlockSpec(memory_space=pl.ANY),
            pl.BlockSpec(memory_space=pl.ANY),
        ],
        out_specs=[
            pl.BlockSpec(memory_space=pl.ANY),
            pl.BlockSpec(memory_space=pl.ANY),
        ],
        out_shape=[
            jax.ShapeDtypeStruct((b, 2 * c + ydim, n_per_b), jnp.float32),
            jax.ShapeDtypeStruct((b * n_per_b, k), jnp.float32),
        ],
        scratch_shapes=[
            pltpu.VMEM((DEPTH, c, nb), jnp.bfloat16),
            pltpu.VMEM((DEPTH, nb, k), jnp.float32),
            pltpu.VMEM((DEPTH, 2 * c + ydim, nb), jnp.float32),
            pltpu.VMEM((c, k), jnp.float32),
            pltpu.VMEM((c, k), jnp.bfloat16),
            pltpu.VMEM((k, c + ydim), jnp.bfloat16),
            pltpu.SemaphoreType.DMA,
            pltpu.SemaphoreType.DMA,
            pltpu.SemaphoreType.DMA((DEPTH,)),
            pltpu.SemaphoreType.DMA((DEPTH,)),
            pltpu.SemaphoreType.DMA((DEPTH,)),
        ],
        compiler_params=pltpu.CompilerParams(
            dimension_semantics=("arbitrary",)),
    )(x3, ft, fl)
    out = out3.reshape(b, 2 * c + ydim, h, w)
    return (out, score)


# speedup vs baseline: 1.0519x; 1.0519x over previous
import functools

import jax
import jax.numpy as jnp
from jax.experimental import pallas as pl
from jax.experimental.pallas import tpu as pltpu

DEPTH = 8


def _x_copies(x_hbm, x_buf, sem_x, t, nb, jblocks, c):
    i = t // jblocks
    j = t % jblocks
    slot = jax.lax.rem(t, DEPTH)
    return [
        pltpu.make_async_copy(
            x_hbm.at[i, :, pl.ds(j * nb, nb)],
            x_buf.at[slot],
            sem_x.at[slot])
    ]


def _score_copies(score_buf, score_hbm, sem_s, t, nb):
    slot = jax.lax.rem(t, DEPTH)
    qr = nb // 2
    return [
        pltpu.make_async_copy(
            score_buf.at[slot, pl.ds(q * qr, qr), :],
            score_hbm.at[pl.ds(t * nb + q * qr, qr), :],
            sem_s.at[slot])
        for q in range(2)
    ]


def _out_copies(out_buf, out_hbm, sem_o, t, nb, jblocks, c, ydim):
    i = t // jblocks
    j = t % jblocks
    slot = jax.lax.rem(t, DEPTH)
    rows = [(0, c), (c, c + ydim)]
    return [
        pltpu.make_async_copy(
            out_buf.at[slot, pl.ds(r0, rn), :],
            out_hbm.at[i, pl.ds(r0, rn), pl.ds(j * nb, nb)],
            sem_o.at[slot])
        for (r0, rn) in rows
    ]


def _body(x_hbm, ft_hbm, fl_hbm, out_hbm, score_hbm,
          x_buf, score_buf, out_buf, ftv, mn, flv,
          sem_ft, sem_fl, sem_x, sem_s, sem_o,
          *, c, k, ydim, nb, jblocks, nsteps):
    t = pl.program_id(0)
    slot = jax.lax.rem(t, DEPTH)

    @pl.when(t == 0)
    def _init():
        cp_ft = pltpu.make_async_copy(ft_hbm, ftv, sem_ft)
        cp_fl = pltpu.make_async_copy(fl_hbm, flv, sem_fl)
        cp_ft.start()
        cp_fl.start()
        for tt in range(3):
            for cp in _x_copies(x_hbm, x_buf, sem_x, tt, nb, jblocks, c):
                cp.start()
        cp_ft.wait()
        cp_fl.wait()
        ft = ftv[...]
        csq = jnp.sum(ft * ft, axis=0, keepdims=True)
        cinv = 1.0 / jnp.maximum(jnp.sqrt(csq), 1e-12)
        mn[...] = (ft * cinv).astype(jnp.bfloat16)

    @pl.when(jnp.logical_and(t >= 1, t + 2 < nsteps))
    def _prefetch():
        for cp in _x_copies(x_hbm, x_buf, sem_x, t + 2, nb, jblocks, c):
            cp.start()

    @pl.when(t >= DEPTH)
    def _drain():
        for cp in _score_copies(score_buf, score_hbm, sem_s, t - DEPTH, nb):
            cp.wait()
        for cp in _out_copies(out_buf, out_hbm, sem_o, t - DEPTH, nb,
                              jblocks, c, ydim):
            cp.wait()

    @pl.when(t < nsteps)
    def _compute():
        for cp in _x_copies(x_hbm, x_buf, sem_x, t, nb, jblocks, c):
            cp.wait()
        xf = x_buf[slot]
        ssq = jnp.sum(xf * xf, axis=0, keepdims=True)
        rinv = 1.0 / jnp.maximum(jnp.sqrt(ssq), 1e-12)
        xn_t = (xf * rinv).astype(jnp.bfloat16)
        s = jax.lax.dot_general(
            xn_t, mn[...],
            dimension_numbers=(((0,), (0,)), ((), ())),
            preferred_element_type=jnp.float32)
        score_buf[slot] = s
        p = jnp.exp(s)
        dinv = 1.0 / jnp.sum(p, axis=1, keepdims=True)
        oxy = jax.lax.dot_general(
            p.astype(jnp.bfloat16), flv[...],
            dimension_numbers=(((1,), (0,)), ((), ())),
            preferred_element_type=jnp.float32)
        oxy = oxy * dinv
        out_buf[slot, :c, :] = xf
        out_buf[slot, c:, :] = oxy.T
        for cp in _score_copies(score_buf, score_hbm, sem_s, t, nb):
            cp.start()
        for cp in _out_copies(out_buf, out_hbm, sem_o, t, nb, jblocks,
                              c, ydim):
            cp.start()


def kernel(x, feat_units, label_units):
    b, c, h, w = x.shape
    k, ydim = label_units.shape[0], label_units.shape[1]
    n_per_b = h * w
    nb = 512 if n_per_b % 512 == 0 else n_per_b
    jblocks = n_per_b // nb
    nsteps = b * jblocks

    x3 = x.reshape(b, c, n_per_b)
    ft = feat_units.T
    fl = jnp.concatenate([feat_units, label_units],
                         axis=1).astype(jnp.bfloat16)

    out3, score = pl.pallas_call(
        functools.partial(_body, c=c, k=k, ydim=ydim, nb=nb,
                          jblocks=jblocks, nsteps=nsteps),
        grid=(nsteps + DEPTH,),
        in_specs=[
            pl.BlockSpec(memory_space=pl.ANY),
            pl.BlockSpec(memory_space=pl.ANY),
            pl.BlockSpec(memory_space=pl.ANY),
        ],
        out_specs=[
            pl.BlockSpec(memory_space=pl.ANY),
            pl.BlockSpec(memory_space=pl.ANY),
        ],
        out_shape=[
            jax.ShapeDtypeStruct((b, 2 * c + ydim, n_per_b), jnp.float32),
            jax.ShapeDtypeStruct((b * n_per_b, k), jnp.float32),
        ],
        scratch_shapes=[
            pltpu.VMEM((DEPTH, c, nb), jnp.float32),
            pltpu.VMEM((DEPTH, nb, k), jnp.float32),
            pltpu.VMEM((DEPTH, 2 * c + ydim, nb), jnp.float32),
            pltpu.VMEM((c, k), jnp.float32),
            pltpu.VMEM((c, k), jnp.bfloat16),
            pltpu.VMEM((k, c + ydim), jnp.bfloat16),
            pltpu.SemaphoreType.DMA,
            pltpu.SemaphoreType.DMA,
            pltpu.SemaphoreType.DMA((DEPTH,)),
            pltpu.SemaphoreType.DMA((DEPTH,)),
            pltpu.SemaphoreType.DMA((DEPTH,)),
        ],
        compiler_params=pltpu.CompilerParams(
            dimension_semantics=("arbitrary",)),
    )(x3, ft, fl)
    out = out3.reshape(b, 2 * c + ydim, h, w)
    return (out, score)


# speedup vs baseline: 1.0990x; 1.0448x over previous
import functools

import jax
import jax.numpy as jnp
from jax.experimental import pallas as pl
from jax.experimental.pallas import tpu as pltpu

DEPTH = 4


def _x_copies(x_hbm, x_buf, sem_x, t, nb, jblocks, c):
    i = t // jblocks
    j = t % jblocks
    slot = jax.lax.rem(t, DEPTH)
    return [
        pltpu.make_async_copy(
            x_hbm.at[i, :, pl.ds(j * nb, nb)],
            x_buf.at[slot],
            sem_x.at[slot])
    ]


def _score_copies(score_buf, score_hbm, sem_s, t, nb):
    slot = jax.lax.rem(t, DEPTH)
    qr = nb // 2
    return [
        pltpu.make_async_copy(
            score_buf.at[slot, pl.ds(q * qr, qr), :],
            score_hbm.at[pl.ds(t * nb + q * qr, qr), :],
            sem_s.at[slot])
        for q in range(2)
    ]


def _out_copies(out_buf, out_hbm, sem_o, t, nb, jblocks, c, ydim):
    i = t // jblocks
    j = t % jblocks
    slot = jax.lax.rem(t, DEPTH)
    rows = [(0, c), (c, c + ydim)]
    return [
        pltpu.make_async_copy(
            out_buf.at[slot, pl.ds(r0, rn), :],
            out_hbm.at[i, pl.ds(r0, rn), pl.ds(j * nb, nb)],
            sem_o.at[slot])
        for (r0, rn) in rows
    ]


def _body(x_hbm, ft_hbm, fl_hbm, out_hbm, score_hbm,
          x_buf, score_buf, out_buf, ftv, mn, flv,
          sem_ft, sem_fl, sem_x, sem_s, sem_o,
          *, c, k, ydim, nb, jblocks, nsteps):
    t = pl.program_id(0)
    slot = jax.lax.rem(t, DEPTH)

    @pl.when(t == 0)
    def _init():
        cp_ft = pltpu.make_async_copy(ft_hbm, ftv, sem_ft)
        cp_fl = pltpu.make_async_copy(fl_hbm, flv, sem_fl)
        cp_ft.start()
        cp_fl.start()
        for tt in range(3):
            for cp in _x_copies(x_hbm, x_buf, sem_x, tt, nb, jblocks, c):
                cp.start()
        cp_ft.wait()
        cp_fl.wait()
        ft = ftv[...]
        csq = jnp.sum(ft * ft, axis=0, keepdims=True)
        cinv = 1.0 / jnp.maximum(jnp.sqrt(csq), 1e-12)
        mn[...] = (ft * cinv).astype(jnp.bfloat16)

    @pl.when(jnp.logical_and(t >= 1, t + 2 < nsteps))
    def _prefetch():
        for cp in _x_copies(x_hbm, x_buf, sem_x, t + 2, nb, jblocks, c):
            cp.start()

    @pl.when(t >= DEPTH)
    def _drain():
        for cp in _score_copies(score_buf, score_hbm, sem_s, t - DEPTH, nb):
            cp.wait()
        for cp in _out_copies(out_buf, out_hbm, sem_o, t - DEPTH, nb,
                              jblocks, c, ydim):
            cp.wait()

    @pl.when(t < nsteps)
    def _compute():
        for cp in _x_copies(x_hbm, x_buf, sem_x, t, nb, jblocks, c):
            cp.wait()
        xf = x_buf[slot]
        ssq = jnp.sum(xf * xf, axis=0, keepdims=True)
        rinv = 1.0 / jnp.maximum(jnp.sqrt(ssq), 1e-12)
        xn_t = (xf * rinv).astype(jnp.bfloat16)
        s = jax.lax.dot_general(
            xn_t, mn[...],
            dimension_numbers=(((0,), (0,)), ((), ())),
            preferred_element_type=jnp.float32)
        score_buf[slot] = s
        p = jnp.exp(s)
        dinv = 1.0 / jnp.sum(p, axis=1, keepdims=True)
        oxy = jax.lax.dot_general(
            p.astype(jnp.bfloat16), flv[...],
            dimension_numbers=(((1,), (0,)), ((), ())),
            preferred_element_type=jnp.float32)
        oxy = oxy * dinv
        out_buf[slot, :c, :] = xf
        out_buf[slot, c:, :] = oxy.T
        for cp in _score_copies(score_buf, score_hbm, sem_s, t, nb):
            cp.start()
        for cp in _out_copies(out_buf, out_hbm, sem_o, t, nb, jblocks,
                              c, ydim):
            cp.start()


def kernel(x, feat_units, label_units):
    b, c, h, w = x.shape
    k, ydim = label_units.shape[0], label_units.shape[1]
    n_per_b = h * w
    nb = 1024 if n_per_b % 1024 == 0 else n_per_b
    jblocks = n_per_b // nb
    nsteps = b * jblocks

    x3 = x.reshape(b, c, n_per_b)
    ft = feat_units.T
    fl = jnp.concatenate([feat_units, label_units],
                         axis=1).astype(jnp.bfloat16)

    out3, score = pl.pallas_call(
        functools.partial(_body, c=c, k=k, ydim=ydim, nb=nb,
                          jblocks=jblocks, nsteps=nsteps),
        grid=(nsteps + DEPTH,),
        in_specs=[
            pl.BlockSpec(memory_space=pl.ANY),
            pl.BlockSpec(memory_space=pl.ANY),
            pl.BlockSpec(memory_space=pl.ANY),
        ],
        out_specs=[
            pl.BlockSpec(memory_space=pl.ANY),
            pl.BlockSpec(memory_space=pl.ANY),
        ],
        out_shape=[
            jax.ShapeDtypeStruct((b, 2 * c + ydim, n_per_b), jnp.float32),
            jax.ShapeDtypeStruct((b * n_per_b, k), jnp.float32),
        ],
        scratch_shapes=[
            pltpu.VMEM((DEPTH, c, nb), jnp.float32),
            pltpu.VMEM((DEPTH, nb, k), jnp.float32),
            pltpu.VMEM((DEPTH, 2 * c + ydim, nb), jnp.float32),
            pltpu.VMEM((c, k), jnp.float32),
            pltpu.VMEM((c, k), jnp.bfloat16),
            pltpu.VMEM((k, c + ydim), jnp.bfloat16),
            pltpu.SemaphoreType.DMA,
            pltpu.SemaphoreType.DMA,
            pltpu.SemaphoreType.DMA((DEPTH,)),
            pltpu.SemaphoreType.DMA((DEPTH,)),
            pltpu.SemaphoreType.DMA((DEPTH,)),
        ],
        compiler_params=pltpu.CompilerParams(
            dimension_semantics=("arbitrary",)),
    )(x3, ft, fl)
    out = out3.reshape(b, 2 * c + ydim, h, w)
    return (out, score)
